# baseline (speedup 1.0000x reference)
import functools

import jax
import jax.numpy as jnp
from jax.experimental import pallas as pl
from jax.experimental.pallas import tpu as pltpu

NUM_EXPERTS = 8
N_EMBD = 1024
EXPERT_DIM = 2048
NUM_TOKENS = 4096
TOP_K = 2

NUM_ASSIGN = NUM_TOKENS * TOP_K
ROW_BLOCK = 128
NBLK = NUM_ASSIGN // ROW_BLOCK + NUM_EXPERTS
NP = NBLK * ROW_BLOCK

SUB = N_EMBD // 128
TOK_BLOCK = 128


def _moe_block_kernel(be_ref, tok_ref, x_ref, w1_ref, w2_ref, yg_ref, xg_scr):
    b = pl.program_id(0)
    base = b * ROW_BLOCK
    for r in range(ROW_BLOCK):
        t = tok_ref[base + r]
        xg_scr[pl.ds(r, 1)] = x_ref[pl.ds(t, 1)]
    xg = xg_scr[...].reshape(ROW_BLOCK, N_EMBD)
    w1e = w1_ref[0]
    h = jax.lax.dot_general(xg, w1e, (((1,), (1,)), ((), ())),
                            preferred_element_type=jnp.float32)
    h = jnp.square(jnp.maximum(h, 0.0))
    w2e = w2_ref[0]
    y = jax.lax.dot_general(h, w2e, (((1,), (1,)), ((), ())),
                            preferred_element_type=jnp.float32)
    yg_ref[...] = y.reshape(ROW_BLOCK, SUB, 128)


def _combine_kernel(pos_ref, ew_ref, yg_ref, out_ref):
    b = pl.program_id(0)
    base = b * TOK_BLOCK
    for r in range(TOK_BLOCK):
        t = base + r
        p0 = pos_ref[2 * t]
        p1 = pos_ref[2 * t + 1]
        w0 = ew_ref[2 * t]
        w1 = ew_ref[2 * t + 1]
        out_ref[pl.ds(r, 1)] = (w0 * yg_ref[pl.ds(p0, 1)]
                                + w1 * yg_ref[pl.ds(p1, 1)])


@jax.jit
def _run(x, expert_indices, expert_weights, w1, w2):
    e = expert_indices.reshape(-1).astype(jnp.int32)
    ew = expert_weights.reshape(-1).astype(jnp.float32)

    oh = (e[:, None] == jnp.arange(NUM_EXPERTS, dtype=jnp.int32)).astype(jnp.int32)
    cum = jnp.cumsum(oh, axis=0)
    counts = cum[-1]
    rank = jnp.take_along_axis(cum, e[:, None], axis=1)[:, 0] - 1
    padded = ((counts + ROW_BLOCK - 1) // ROW_BLOCK) * ROW_BLOCK
    starts = jnp.concatenate([jnp.zeros(1, jnp.int32), jnp.cumsum(padded)]).astype(jnp.int32)
    pos = starts[e] + rank

    tok = jnp.arange(NUM_ASSIGN, dtype=jnp.int32) // TOP_K
    src_token = jnp.zeros((NP,), jnp.int32).at[pos].set(tok)

    blk_start = jnp.arange(NBLK, dtype=jnp.int32) * ROW_BLOCK
    block_expert = jnp.clip(
        jnp.searchsorted(starts[1:], blk_start, side='right').astype(jnp.int32),
        0, NUM_EXPERTS - 1)

    x3 = x.reshape(NUM_TOKENS, SUB, 128)

    yg = pl.pallas_call(
        _moe_block_kernel,
        grid_spec=pltpu.PrefetchScalarGridSpec(
            num_scalar_prefetch=2,
            grid=(NBLK,),
            in_specs=[
                pl.BlockSpec((NUM_TOKENS, SUB, 128), lambda b, be, tk: (0, 0, 0)),
                pl.BlockSpec((1, EXPERT_DIM, N_EMBD), lambda b, be, tk: (be[b], 0, 0)),
                pl.BlockSpec((1, N_EMBD, EXPERT_DIM), lambda b, be, tk: (be[b], 0, 0)),
            ],
            out_specs=pl.BlockSpec((ROW_BLOCK, SUB, 128), lambda b, be, tk: (b, 0, 0)),
            scratch_shapes=[pltpu.VMEM((ROW_BLOCK, SUB, 128), jnp.float32)],
        ),
        out_shape=jax.ShapeDtypeStruct((NP, SUB, 128), jnp.float32),
        compiler_params=pltpu.CompilerParams(
            dimension_semantics=("arbitrary",),
        ),
    )(block_expert, src_token, x3, w1, w2)

    out3 = pl.pallas_call(
        _combine_kernel,
        grid_spec=pltpu.PrefetchScalarGridSpec(
            num_scalar_prefetch=2,
            grid=(NUM_TOKENS // TOK_BLOCK,),
            in_specs=[
                pl.BlockSpec((NP, SUB, 128), lambda b, ps, w: (0, 0, 0)),
            ],
            out_specs=pl.BlockSpec((TOK_BLOCK, SUB, 128), lambda b, ps, w: (b, 0, 0)),
        ),
        out_shape=jax.ShapeDtypeStruct((NUM_TOKENS, SUB, 128), jnp.float32),
        compiler_params=pltpu.CompilerParams(
            dimension_semantics=("arbitrary",),
        ),
    )(pos, ew, yg)

    return out3.reshape(NUM_TOKENS, N_EMBD)


def kernel(x, expert_indices, expert_weights, w1, w2):
    return _run(x, expert_indices, expert_weights, w1, w2)
